# Initial kernel scaffold; baseline (speedup 1.0000x reference)
import jax
import jax.numpy as jnp
from jax.experimental import pallas as pl


def kernel(input, emb_table, W_ih, b_ih, W_hh, b_hh, W_out, b_out):
    raise NotImplementedError("write your pallas kernel here")


# speedup vs baseline: 3.1227x; 3.1227x over previous
import functools

import jax
import jax.numpy as jnp
from jax import lax
from jax.experimental import pallas as pl
from jax.experimental.pallas import tpu as pltpu
from jax.experimental.pallas import tpu_sc as plsc

VOCAB = 10000
EMB = 512
HID = 512
SEQ = 128
BATCH = 32
NTOK = SEQ * BATCH

_NC = 2
_NS = 16
_NW = _NC * _NS
_TOK_PER_W = NTOK // _NW


def _gather_body(table_hbm, idx_hbm, out_hbm, idx_v, rows_v, sem):
    wid = lax.axis_index("s") * _NC + lax.axis_index("c")
    base = wid * _TOK_PER_W
    pltpu.sync_copy(idx_hbm.at[pl.ds(base, _TOK_PER_W)], idx_v)
    pltpu.async_copy(table_hbm.at[idx_v], rows_v, sem).wait()
    pltpu.sync_copy(rows_v, out_hbm.at[pl.ds(base, _TOK_PER_W)])


@functools.lru_cache(maxsize=1)
def _sc_gather():
    return pl.kernel(
        _gather_body,
        out_type=jax.ShapeDtypeStruct((NTOK, EMB), jnp.float32),
        mesh=plsc.VectorSubcoreMesh(core_axis_name="c", subcore_axis_name="s"),
        scratch_types=[
            pltpu.VMEM((_TOK_PER_W,), jnp.int32),
            pltpu.VMEM((_TOK_PER_W, EMB), jnp.float32),
            pltpu.SemaphoreType.DMA,
        ],
    )


def _rnn_body(x_ref, wih_ref, whh_ref, b_ref, out_ref, hT_ref):
    xw = lax.dot_general(
        x_ref[:], wih_ref[:],
        (((1,), (1,)), ((), ())),
        preferred_element_type=jnp.float32,
    ) + b_ref[:]
    out_ref[:] = xw.reshape(SEQ, BATCH, HID)

    def step(t, h):
        pre = out_ref[t] + lax.dot_general(
            h, whh_ref[:],
            (((1,), (1,)), ((), ())),
            preferred_element_type=jnp.float32,
        )
        h_new = jnp.tanh(pre)
        out_ref[t] = h_new
        return h_new

    h0 = jnp.zeros((BATCH, HID), dtype=jnp.float32)
    hT = lax.fori_loop(0, SEQ, step, h0)
    hT_ref[:] = hT


_rnn = pl.pallas_call(
    _rnn_body,
    out_shape=(
        jax.ShapeDtypeStruct((SEQ, BATCH, HID), jnp.float32),
        jax.ShapeDtypeStruct((BATCH, HID), jnp.float32),
    ),
)


_VT = 1024
_NV = (VOCAB + _VT - 1) // _VT


def _proj_body(outs_ref, w_ref, b_ref, o_ref):
    o_ref[:] = lax.dot_general(
        outs_ref[:], w_ref[:],
        (((1,), (1,)), ((), ())),
        preferred_element_type=jnp.float32,
    ) + b_ref[:]


_proj = pl.pallas_call(
    _proj_body,
    grid=(_NV,),
    in_specs=[
        pl.BlockSpec((NTOK, HID), lambda j: (0, 0)),
        pl.BlockSpec((_VT, HID), lambda j: (j, 0)),
        pl.BlockSpec((1, _VT), lambda j: (0, j)),
    ],
    out_specs=pl.BlockSpec((NTOK, _VT), lambda j: (0, j)),
    out_shape=jax.ShapeDtypeStruct((NTOK, VOCAB), jnp.float32),
)


def kernel(input, emb_table, W_ih, b_ih, W_hh, b_hh, W_out, b_out):
    idx = input.reshape(-1).astype(jnp.int32)
    x = _sc_gather()(emb_table, idx)
    b = (b_ih + b_hh).reshape(1, HID)
    outs, hT = _rnn(x, W_ih, W_hh, b)
    logits = _proj(outs.reshape(NTOK, HID), W_out, b_out.reshape(1, VOCAB))
    return (logits, hT[None, :, :])


# speedup vs baseline: 3.1557x; 1.0106x over previous
import functools

import jax
import jax.numpy as jnp
from jax import lax
from jax.experimental import pallas as pl
from jax.experimental.pallas import tpu as pltpu
from jax.experimental.pallas import tpu_sc as plsc

VOCAB = 10000
EMB = 512
HID = 512
SEQ = 128
BATCH = 32
NTOK = SEQ * BATCH

_NC = 2
_NS = 16
_NW = _NC * _NS
_TOK_PER_W = NTOK // _NW


def _gather_body(table_hbm, idx_hbm, out_hbm, idx_v, rows_v, sem):
    wid = lax.axis_index("s") * _NC + lax.axis_index("c")
    base = wid * _TOK_PER_W
    pltpu.sync_copy(idx_hbm.at[pl.ds(base, _TOK_PER_W)], idx_v)
    pltpu.async_copy(table_hbm.at[idx_v], rows_v, sem).wait()
    pltpu.sync_copy(rows_v, out_hbm.at[pl.ds(base, _TOK_PER_W)])


@functools.lru_cache(maxsize=1)
def _sc_gather():
    return pl.kernel(
        _gather_body,
        out_type=jax.ShapeDtypeStruct((NTOK, EMB), jnp.float32),
        mesh=plsc.VectorSubcoreMesh(core_axis_name="c", subcore_axis_name="s"),
        scratch_types=[
            pltpu.VMEM((_TOK_PER_W,), jnp.int32),
            pltpu.VMEM((_TOK_PER_W, EMB), jnp.float32),
            pltpu.SemaphoreType.DMA,
        ],
    )


def _rnn_body(x_ref, wih_ref, whh_ref, b_ref, out_ref, hT_ref, xw_ref):
    xw = lax.dot_general(
        x_ref[:], wih_ref[:],
        (((1,), (1,)), ((), ())),
        preferred_element_type=jnp.float32,
    ) + b_ref[:]
    xw_ref[:] = xw.reshape(SEQ, BATCH, HID)

    def step(t, h):
        pre = xw_ref[t] + lax.dot_general(
            h, whh_ref[:],
            (((1,), (1,)), ((), ())),
            preferred_element_type=jnp.float32,
        )
        h_new = jnp.tanh(pre)
        out_ref[t] = h_new.astype(jnp.bfloat16)
        return h_new

    h0 = jnp.zeros((BATCH, HID), dtype=jnp.float32)
    hT = lax.fori_loop(0, SEQ, step, h0)
    hT_ref[:] = hT


_rnn = pl.pallas_call(
    _rnn_body,
    out_shape=(
        jax.ShapeDtypeStruct((SEQ, BATCH, HID), jnp.bfloat16),
        jax.ShapeDtypeStruct((BATCH, HID), jnp.float32),
    ),
    scratch_shapes=[pltpu.VMEM((SEQ, BATCH, HID), jnp.float32)],
)


_VT = 1024
_NV = (VOCAB + _VT - 1) // _VT


def _proj_body(outs_ref, w_ref, b_ref, o_ref):
    o_ref[:] = lax.dot_general(
        outs_ref[:], w_ref[:].astype(jnp.bfloat16),
        (((1,), (1,)), ((), ())),
        preferred_element_type=jnp.float32,
    ) + b_ref[:]


_proj = pl.pallas_call(
    _proj_body,
    grid=(_NV,),
    in_specs=[
        pl.BlockSpec((NTOK, HID), lambda j: (0, 0)),
        pl.BlockSpec((_VT, HID), lambda j: (j, 0)),
        pl.BlockSpec((1, _VT), lambda j: (0, j)),
    ],
    out_specs=pl.BlockSpec((NTOK, _VT), lambda j: (0, j)),
    out_shape=jax.ShapeDtypeStruct((NTOK, VOCAB), jnp.float32),
)


def kernel(input, emb_table, W_ih, b_ih, W_hh, b_hh, W_out, b_out):
    idx = input.reshape(-1).astype(jnp.int32)
    x = _sc_gather()(emb_table, idx)
    b = (b_ih + b_hh).reshape(1, HID)
    outs, hT = _rnn(x, W_ih, W_hh, b)
    logits = _proj(outs.reshape(NTOK, HID), W_out, b_out.reshape(1, VOCAB))
    return (logits, hT[None, :, :])


# speedup vs baseline: 3.3329x; 1.0562x over previous
import functools

import jax
import jax.numpy as jnp
from jax import lax
from jax.experimental import pallas as pl
from jax.experimental.pallas import tpu as pltpu
from jax.experimental.pallas import tpu_sc as plsc

VOCAB = 10000
EMB = 512
HID = 512
SEQ = 128
BATCH = 32
NTOK = SEQ * BATCH

_NC = 2
_NS = 16
_NW = _NC * _NS
_TOK_PER_W = NTOK // _NW


def _gather_body(table_hbm, idx_hbm, out_hbm, idx_v, rows_v, sem):
    wid = lax.axis_index("s") * _NC + lax.axis_index("c")
    base = wid * _TOK_PER_W
    pltpu.sync_copy(idx_hbm.at[pl.ds(base, _TOK_PER_W)], idx_v)
    pltpu.async_copy(table_hbm.at[idx_v], rows_v, sem).wait()
    pltpu.sync_copy(rows_v, out_hbm.at[pl.ds(base, _TOK_PER_W)])


@functools.lru_cache(maxsize=1)
def _sc_gather():
    return pl.kernel(
        _gather_body,
        out_type=jax.ShapeDtypeStruct((NTOK, EMB), jnp.float32),
        mesh=plsc.VectorSubcoreMesh(core_axis_name="c", subcore_axis_name="s"),
        scratch_types=[
            pltpu.VMEM((_TOK_PER_W,), jnp.int32),
            pltpu.VMEM((_TOK_PER_W, EMB), jnp.float32),
            pltpu.SemaphoreType.DMA,
        ],
    )


_TBLK = 8
_RBLK = _TBLK * BATCH
_NBLK = SEQ // _TBLK


def _fused_body(x_ref, wih_ref, whh_ref, b_ref, wout_ref, bout_ref,
                o_ref, hT_ref, h_ref, wbf_ref):
    i = pl.program_id(0)

    @pl.when(i == 0)
    def _init():
        h_ref[:] = jnp.zeros((BATCH, HID), jnp.float32)
        wbf_ref[:] = wout_ref[:].astype(jnp.bfloat16)

    xw = lax.dot_general(
        x_ref[:], wih_ref[:],
        (((1,), (1,)), ((), ())),
        preferred_element_type=jnp.float32,
    ) + b_ref[:]
    xw3 = xw.reshape(_TBLK, BATCH, HID)

    h = h_ref[:]
    hs = []
    for t in range(_TBLK):
        pre = xw3[t] + lax.dot_general(
            h, whh_ref[:],
            (((1,), (1,)), ((), ())),
            preferred_element_type=jnp.float32,
        )
        h = jnp.tanh(pre)
        hs.append(h)
    h_ref[:] = h
    hT_ref[:] = h

    outs = jnp.stack(hs).reshape(_RBLK, HID).astype(jnp.bfloat16)
    o_ref[:] = lax.dot_general(
        outs, wbf_ref[:],
        (((1,), (1,)), ((), ())),
        preferred_element_type=jnp.float32,
    ) + bout_ref[:]


_fused = pl.pallas_call(
    _fused_body,
    grid=(_NBLK,),
    in_specs=[
        pl.BlockSpec((_RBLK, EMB), lambda i: (i, 0)),
        pl.BlockSpec((HID, EMB), lambda i: (0, 0)),
        pl.BlockSpec((HID, HID), lambda i: (0, 0)),
        pl.BlockSpec((1, HID), lambda i: (0, 0)),
        pl.BlockSpec((VOCAB, HID), lambda i: (0, 0)),
        pl.BlockSpec((1, VOCAB), lambda i: (0, 0)),
    ],
    out_specs=(
        pl.BlockSpec((_RBLK, VOCAB), lambda i: (i, 0)),
        pl.BlockSpec((BATCH, HID), lambda i: (0, 0)),
    ),
    out_shape=(
        jax.ShapeDtypeStruct((NTOK, VOCAB), jnp.float32),
        jax.ShapeDtypeStruct((BATCH, HID), jnp.float32),
    ),
    scratch_shapes=[
        pltpu.VMEM((BATCH, HID), jnp.float32),
        pltpu.VMEM((VOCAB, HID), jnp.bfloat16),
    ],
)


def kernel(input, emb_table, W_ih, b_ih, W_hh, b_hh, W_out, b_out):
    idx = input.reshape(-1).astype(jnp.int32)
    x = _sc_gather()(emb_table, idx)
    b = (b_ih + b_hh).reshape(1, HID)
    logits, hT = _fused(x, W_ih, W_hh, b, W_out, b_out.reshape(1, VOCAB))
    return (logits, hT[None, :, :])
